# baseline (speedup 1.0000x reference)
import functools

import jax
import jax.numpy as jnp
from jax import lax
from jax.experimental import pallas as pl
from jax.experimental.pallas import tpu as pltpu
from jax.experimental.pallas import tpu_sc as plsc

_D = 128
_WINDOW = 128
_LANES = 16


def _prep_body(dt_ref, st_ref, t3_ref):
    dt = dt_ref[...]
    st = st_ref[...]
    t3_ref[...] = dt[:, None, :] + st[None, :, :]


def _sc_gather(table, d_ids2, s_ids2, n):
    mesh = plsc.VectorSubcoreMesh(core_axis_name="c", subcore_axis_name="s")

    @functools.partial(
        pl.kernel,
        out_type=jax.ShapeDtypeStruct((n, _D), jnp.float32),
        mesh=mesh,
        scratch_types=[
            pltpu.VMEM_SHARED((1000, _D), jnp.float32),
            pltpu.VMEM((_WINDOW,), jnp.int32),
        ],
    )
    def k(tbl_hbm, d_hbm, s_hbm, out_hbm, tbl_sh, idx_v):
        @pl.when(lax.axis_index("s") == 0)
        def _():
            pltpu.sync_copy(tbl_hbm, tbl_sh)

        plsc.subcore_barrier()

        def body(d_v, s_v, o_vmem):
            @pl.loop(0, _WINDOW, step=_LANES)
            def _(j):
                dd = d_v[0, pl.ds(j, _LANES)]
                ss = s_v[0, pl.ds(j, _LANES)]
                idx_v[pl.ds(j, _LANES)] = dd * 50 + ss

            pltpu.sync_copy(tbl_sh.at[idx_v], o_vmem)

        pltpu.emit_pipeline(
            body,
            grid=(n // _WINDOW,),
            in_specs=[
                pl.BlockSpec((1, _WINDOW), lambda i: (i, 0)),
                pl.BlockSpec((1, _WINDOW), lambda i: (i, 0)),
            ],
            out_specs=[pl.BlockSpec((_WINDOW, _D), lambda i: (i, 0))],
            core_axis_name=("c", "s"),
            dimension_semantics=(pltpu.PARALLEL,),
        )(d_hbm, s_hbm, out_hbm)

    return k(table, d_ids2, s_ids2)


def kernel(depth_ids, subtree_ids, depth_table, subtree_table):
    b, sq = depth_ids.shape
    nd, d_model = depth_table.shape
    ns = subtree_table.shape[0]
    n = b * sq

    d_ids2 = depth_ids.reshape(n // _WINDOW, _WINDOW).astype(jnp.int32)
    s_ids2 = subtree_ids.reshape(n // _WINDOW, _WINDOW).astype(jnp.int32)

    t3 = pl.pallas_call(
        _prep_body,
        out_shape=jax.ShapeDtypeStruct((nd, ns, d_model), jnp.float32),
    )(depth_table, subtree_table)

    table = t3.reshape(nd * ns, d_model)

    out = _sc_gather(table, d_ids2, s_ids2, n)
    return out.reshape(b, sq, d_model)
